# baseline (speedup 1.0000x reference)
import functools

import jax
import jax.numpy as jnp
from jax import lax
from jax.experimental import pallas as pl
from jax.experimental.pallas import tpu as pltpu
from jax.experimental.pallas import tpu_sc as plsc

N = 10000
E = 320000
D = 128
DH = D // 2
NC = 2
NS = 16
NW = NC * NS
CHUNK = 128
CPW = 80
NCHUNKS = NW * CPW
E_PAD = NCHUNKS * CHUNK
TRASH = 112
NP = N + TRASH
STRIPE = NP // NS
DEGW = 8

_MESH = plsc.VectorSubcoreMesh(
    core_axis_name="c", subcore_axis_name="s", num_cores=NC, num_subcores=NS
)
_SC_PARAMS = pltpu.CompilerParams(use_tc_tiling_on_sc=False)


def _deg_body(dst2d_hbm, zeros8_hbm, out_hbm, dst_st, ones_st, deg_sh, dsem):
    c = lax.axis_index("c")
    s = lax.axis_index("s")
    wid = s * NC + c
    r0 = s * STRIPE

    pltpu.sync_copy(zeros8_hbm.at[pl.ds(r0, STRIPE)], deg_sh.at[pl.ds(r0, STRIPE)])
    pltpu.sync_copy(dst2d_hbm.at[pl.ds(wid * CPW, CPW)], dst_st)
    for r in range(CHUNK // 2):
        ones_st[pl.ds(2 * r, 2), :] = jnp.ones((2, DEGW), jnp.float32)
    plsc.subcore_barrier()

    k = 8

    def grp(g, carry):
        for b in range(k):
            pltpu.async_copy(
                ones_st, deg_sh.at[dst_st.at[g * k + b]], dsem, add=True
            )
        for b in range(k):
            pltpu.make_async_copy(
                zeros8_hbm.at[pl.ds(0, CHUNK)], ones_st, dsem
            ).wait()
        return carry

    lax.fori_loop(0, CPW // k, grp, 0)
    plsc.subcore_barrier()
    pltpu.sync_copy(
        deg_sh.at[pl.ds(r0, STRIPE)], out_hbm.at[c, pl.ds(r0, STRIPE)]
    )


_deg_kernel = functools.partial(
    pl.kernel,
    out_type=jax.ShapeDtypeStruct((NC, NP, DEGW), jnp.float32),
    mesh=_MESH,
    compiler_params=_SC_PARAMS,
    scratch_types=[
        pltpu.VMEM((CPW, CHUNK), jnp.int32),
        pltpu.VMEM((CHUNK, DEGW), jnp.float32),
        pltpu.VMEM_SHARED((NP, DEGW), jnp.float32),
        pltpu.SemaphoreType.DMA,
    ],
)(_deg_body)


def _make_scatter_body(width, nbuf, npass):
    ngrp = CPW // nbuf

    def body(*refs):
        g_hbms = refs[:npass]
        src2d_hbm, dst2d_hbm, zeros_hbm, out_hbm = refs[npass:npass + 4]
        src_st, dst_st, rowbuf, s_sh, gsem, ssem = refs[npass + 4:]

        c = lax.axis_index("c")
        s = lax.axis_index("s")
        wid = s * NC + c
        r0 = s * STRIPE

        pltpu.sync_copy(src2d_hbm.at[pl.ds(wid * CPW, CPW)], src_st)
        pltpu.sync_copy(dst2d_hbm.at[pl.ds(wid * CPW, CPW)], dst_st)

        for half in range(npass):
            g_hbm = g_hbms[half]
            pltpu.sync_copy(
                zeros_hbm.at[pl.ds(r0, STRIPE)], s_sh.at[pl.ds(r0, STRIPE)]
            )
            plsc.subcore_barrier()

            for b in range(nbuf):
                pltpu.async_copy(
                    g_hbm.at[src_st.at[b]], rowbuf.at[b], gsem.at[b]
                )

            def grp(g, carry):
                for b in range(nbuf):
                    j = g * nbuf + b
                    pltpu.make_async_copy(
                        g_hbm.at[src_st.at[b]], rowbuf.at[b], gsem.at[b]
                    ).wait()
                    pltpu.async_copy(
                        rowbuf.at[b], s_sh.at[dst_st.at[j]], ssem.at[b],
                        add=True,
                    )
                for b in range(nbuf):
                    j2 = (g + 1) * nbuf + b
                    pltpu.make_async_copy(
                        g_hbm.at[src_st.at[b]], rowbuf.at[b], ssem.at[b]
                    ).wait()
                    pltpu.async_copy(
                        g_hbm.at[src_st.at[j2]], rowbuf.at[b], gsem.at[b]
                    )
                return carry

            lax.fori_loop(0, ngrp - 1, grp, 0)

            for b in range(nbuf):
                j = (ngrp - 1) * nbuf + b
                pltpu.make_async_copy(
                    g_hbm.at[src_st.at[b]], rowbuf.at[b], gsem.at[b]
                ).wait()
                pltpu.async_copy(
                    rowbuf.at[b], s_sh.at[dst_st.at[j]], ssem.at[b], add=True
                )
            for b in range(nbuf):
                pltpu.make_async_copy(
                    g_hbm.at[src_st.at[b]], rowbuf.at[b], ssem.at[b]
                ).wait()

            plsc.subcore_barrier()
            pltpu.sync_copy(
                s_sh.at[pl.ds(r0, STRIPE)],
                out_hbm.at[half, c, pl.ds(r0, STRIPE)],
            )

    return body


def _make_scatter_kernel(width, nbuf, npass):
    return functools.partial(
        pl.kernel,
        out_type=jax.ShapeDtypeStruct((npass, NC, NP, width), jnp.float32),
        mesh=_MESH,
        compiler_params=_SC_PARAMS,
        scratch_types=[
            pltpu.VMEM((CPW, CHUNK), jnp.int32),
            pltpu.VMEM((CPW, CHUNK), jnp.int32),
            pltpu.VMEM((nbuf, CHUNK, width), jnp.float32),
            pltpu.VMEM_SHARED((NP, width), jnp.float32),
            pltpu.SemaphoreType.DMA((nbuf,)),
            pltpu.SemaphoreType.DMA((nbuf,)),
        ],
    )(_make_scatter_body(width, nbuf, npass))


_scatter_half = _make_scatter_kernel(DH, 8, 2)


B = 1000

_DEG_SPEC = pl.BlockSpec((NC, B, 1), lambda i: (0, i, 0))
_ROW_SPEC = pl.BlockSpec((B, D), lambda i: (i, 0))
_HALF_SPEC = pl.BlockSpec((B, DH), lambda i: (i, 0))
_MAT_SPEC = pl.BlockSpec((D, D), lambda i: (0, 0))
_BIAS_SPEC = pl.BlockSpec((1, D), lambda i: (0, 0))
_SF_SPEC = pl.BlockSpec((1, NC, B, D), lambda i: (0, 0, i, 0))
_SH_SPEC = pl.BlockSpec((2, NC, B, DH), lambda i: (0, 0, i, 0))


def _dinv(deg_ref):
    return lax.rsqrt(deg_ref[0] + deg_ref[1] + 1.0)


def _tc1_body(deg_ref, x_ref, w_ref, glo_ref, ghi_ref):
    h = jnp.dot(x_ref[...], w_ref[...], preferred_element_type=jnp.float32,
                precision=lax.Precision.HIGHEST)
    g = h * _dinv(deg_ref)
    glo_ref[...] = g[:, :DH]
    ghi_ref[...] = g[:, DH:]


def _comb1_body(deg_ref, s_ref, glo_ref, ghi_ref, b_ref, w_ref,
                gnlo_ref, gnhi_ref):
    dinv = _dinv(deg_ref)
    tot = jnp.concatenate(
        [s_ref[0, 0] + s_ref[0, 1] + glo_ref[...],
         s_ref[1, 0] + s_ref[1, 1] + ghi_ref[...]],
        axis=1,
    )
    t = tot * dinv + b_ref[...]
    u = jnp.maximum(t, 0.0)
    h = jnp.dot(u, w_ref[...], preferred_element_type=jnp.float32,
                precision=lax.Precision.HIGHEST)
    gn = h * dinv
    gnlo_ref[...] = gn[:, :DH]
    gnhi_ref[...] = gn[:, DH:]


def _comb2_body(deg_ref, s_ref, glo_ref, ghi_ref, b_ref, out_ref):
    tot = jnp.concatenate(
        [s_ref[0, 0] + s_ref[0, 1] + glo_ref[...],
         s_ref[1, 0] + s_ref[1, 1] + ghi_ref[...]],
        axis=1,
    )
    out_ref[...] = tot * _dinv(deg_ref) + b_ref[...]


_tc1 = pl.pallas_call(
    _tc1_body,
    grid=(N // B,),
    in_specs=[_DEG_SPEC, _ROW_SPEC, _MAT_SPEC],
    out_specs=[_HALF_SPEC, _HALF_SPEC],
    out_shape=[
        jax.ShapeDtypeStruct((N, DH), jnp.float32),
        jax.ShapeDtypeStruct((N, DH), jnp.float32),
    ],
)

_comb1 = pl.pallas_call(
    _comb1_body,
    grid=(N // B,),
    in_specs=[_DEG_SPEC, _SH_SPEC, _HALF_SPEC, _HALF_SPEC, _BIAS_SPEC,
              _MAT_SPEC],
    out_specs=[_HALF_SPEC, _HALF_SPEC],
    out_shape=[
        jax.ShapeDtypeStruct((N, DH), jnp.float32),
        jax.ShapeDtypeStruct((N, DH), jnp.float32),
    ],
)

_comb2 = pl.pallas_call(
    _comb2_body,
    grid=(N // B,),
    in_specs=[_DEG_SPEC, _SH_SPEC, _HALF_SPEC, _HALF_SPEC, _BIAS_SPEC],
    out_specs=_ROW_SPEC,
    out_shape=jax.ShapeDtypeStruct((N, D), jnp.float32),
)


def kernel(x, edge_index, W1, b1, W2, b2):
    src = edge_index[0].astype(jnp.int32)
    dst = edge_index[1].astype(jnp.int32)
    pad = E_PAD - E
    fill = jnp.arange(pad, dtype=jnp.int32)
    src_p = jnp.concatenate([src, fill % 16])
    dst_p = jnp.concatenate([dst, N + (fill % TRASH)])
    src2d = src_p.reshape(NCHUNKS, CHUNK)
    dst2d = dst_p.reshape(NCHUNKS, CHUNK)

    zeros8 = jnp.zeros((NP, DEGW), jnp.float32)
    zerosH = jnp.zeros((NP, DH), jnp.float32)

    deg = _deg_kernel(dst2d, zeros8)
    degc = deg[:, :, :1]

    g1lo, g1hi = _tc1(degc, x, W1)
    s1 = _scatter_half(g1lo, g1hi, src2d, dst2d, zerosH)
    g2lo, g2hi = _comb1(degc, s1, g1lo, g1hi, b1.reshape(1, D), W2)
    s2 = _scatter_half(g2lo, g2hi, src2d, dst2d, zerosH)
    return _comb2(degc, s2, g2lo, g2hi, b2.reshape(1, D))


# speedup vs baseline: 1.0219x; 1.0219x over previous
import functools

import jax
import jax.numpy as jnp
from jax import lax
from jax.experimental import pallas as pl
from jax.experimental.pallas import tpu as pltpu
from jax.experimental.pallas import tpu_sc as plsc

N = 10000
E = 320000
D = 128
DH = D // 2
NC = 2
NS = 16
NW = NC * NS
CHUNK = 128
CPW = 80
NCHUNKS = NW * CPW
E_PAD = NCHUNKS * CHUNK
TRASH = 112
NP = N + TRASH
STRIPE = NP // NS
DEGW = 8

_MESH = plsc.VectorSubcoreMesh(
    core_axis_name="c", subcore_axis_name="s", num_cores=NC, num_subcores=NS
)
_SC_PARAMS = pltpu.CompilerParams(use_tc_tiling_on_sc=False)


def _deg_body(dst2d_hbm, zeros8_hbm, out_hbm, dst_st, ones_st, deg_sh, dsem):
    c = lax.axis_index("c")
    s = lax.axis_index("s")
    wid = s * NC + c
    r0 = s * STRIPE

    pltpu.sync_copy(zeros8_hbm.at[pl.ds(r0, STRIPE)], deg_sh.at[pl.ds(r0, STRIPE)])
    pltpu.sync_copy(dst2d_hbm.at[pl.ds(wid * CPW, CPW)], dst_st)
    for r in range(CHUNK // 2):
        ones_st[pl.ds(2 * r, 2), :] = jnp.ones((2, DEGW), jnp.float32)
    plsc.subcore_barrier()

    k = 8

    def grp(g, carry):
        for b in range(k):
            pltpu.async_copy(
                ones_st, deg_sh.at[dst_st.at[g * k + b]], dsem, add=True
            )
        for b in range(k):
            pltpu.make_async_copy(
                zeros8_hbm.at[pl.ds(0, CHUNK)], ones_st, dsem
            ).wait()
        return carry

    lax.fori_loop(0, CPW // k, grp, 0)
    plsc.subcore_barrier()
    pltpu.sync_copy(
        deg_sh.at[pl.ds(r0, STRIPE)], out_hbm.at[c, pl.ds(r0, STRIPE)]
    )


_deg_kernel = functools.partial(
    pl.kernel,
    out_type=jax.ShapeDtypeStruct((NC, NP, DEGW), jnp.float32),
    mesh=_MESH,
    compiler_params=_SC_PARAMS,
    scratch_types=[
        pltpu.VMEM((CPW, CHUNK), jnp.int32),
        pltpu.VMEM((CHUNK, DEGW), jnp.float32),
        pltpu.VMEM_SHARED((NP, DEGW), jnp.float32),
        pltpu.SemaphoreType.DMA,
    ],
)(_deg_body)


def _make_scatter_body(width, nbuf, npass):
    ngrp = CPW // nbuf

    def body(*refs):
        g_hbms = refs[:npass]
        src2d_hbm, dst2d_hbm, zeros_hbm, out_hbm = refs[npass:npass + 4]
        src_st, dst_st, rowbuf, s_sh, gsem, ssem = refs[npass + 4:]

        c = lax.axis_index("c")
        s = lax.axis_index("s")
        wid = s * NC + c
        r0 = s * STRIPE

        pltpu.sync_copy(src2d_hbm.at[pl.ds(wid * CPW, CPW)], src_st)
        pltpu.sync_copy(dst2d_hbm.at[pl.ds(wid * CPW, CPW)], dst_st)

        for half in range(npass):
            g_hbm = g_hbms[half]
            pltpu.sync_copy(
                zeros_hbm.at[pl.ds(r0, STRIPE)], s_sh.at[pl.ds(r0, STRIPE)]
            )
            plsc.subcore_barrier()

            for b in range(nbuf):
                pltpu.async_copy(
                    g_hbm.at[src_st.at[b]], rowbuf.at[b], gsem.at[b]
                )

            def grp(g, carry):
                for b in range(nbuf):
                    j = g * nbuf + b
                    pltpu.make_async_copy(
                        g_hbm.at[src_st.at[b]], rowbuf.at[b], gsem.at[b]
                    ).wait()
                    pltpu.async_copy(
                        rowbuf.at[b], s_sh.at[dst_st.at[j]], ssem.at[b],
                        add=True,
                    )
                for b in range(nbuf):
                    j2 = (g + 1) * nbuf + b
                    pltpu.make_async_copy(
                        g_hbm.at[src_st.at[b]], rowbuf.at[b], ssem.at[b]
                    ).wait()
                    pltpu.async_copy(
                        g_hbm.at[src_st.at[j2]], rowbuf.at[b], gsem.at[b]
                    )
                return carry

            lax.fori_loop(0, ngrp - 1, grp, 0)

            for b in range(nbuf):
                j = (ngrp - 1) * nbuf + b
                pltpu.make_async_copy(
                    g_hbm.at[src_st.at[b]], rowbuf.at[b], gsem.at[b]
                ).wait()
                pltpu.async_copy(
                    rowbuf.at[b], s_sh.at[dst_st.at[j]], ssem.at[b], add=True
                )
            for b in range(nbuf):
                pltpu.make_async_copy(
                    g_hbm.at[src_st.at[b]], rowbuf.at[b], ssem.at[b]
                ).wait()

            plsc.subcore_barrier()
            pltpu.sync_copy(
                s_sh.at[pl.ds(r0, STRIPE)],
                out_hbm.at[half, c, pl.ds(r0, STRIPE)],
            )

    return body


def _make_scatter_kernel(width, nbuf, npass):
    return functools.partial(
        pl.kernel,
        out_type=jax.ShapeDtypeStruct((npass, NC, NP, width), jnp.float32),
        mesh=_MESH,
        compiler_params=_SC_PARAMS,
        scratch_types=[
            pltpu.VMEM((CPW, CHUNK), jnp.int32),
            pltpu.VMEM((CPW, CHUNK), jnp.int32),
            pltpu.VMEM((nbuf, CHUNK, width), jnp.float32),
            pltpu.VMEM_SHARED((NP, width), jnp.float32),
            pltpu.SemaphoreType.DMA((nbuf,)),
            pltpu.SemaphoreType.DMA((nbuf,)),
        ],
    )(_make_scatter_body(width, nbuf, npass))


_scatter_half = _make_scatter_kernel(DH, 8, 2)


B = 1000

_DEG_SPEC = pl.BlockSpec((NC, B, 1), lambda i: (0, i, 0))
_ROW_SPEC = pl.BlockSpec((B, D), lambda i: (i, 0))
_HALF_SPEC = pl.BlockSpec((B, DH), lambda i: (i, 0))
_MAT_SPEC = pl.BlockSpec((D, D), lambda i: (0, 0))
_BIAS_SPEC = pl.BlockSpec((1, D), lambda i: (0, 0))
_SF_SPEC = pl.BlockSpec((1, NC, B, D), lambda i: (0, 0, i, 0))
_SH_SPEC = pl.BlockSpec((2, NC, B, DH), lambda i: (0, 0, i, 0))


def _dinv(deg_ref):
    return lax.rsqrt(deg_ref[0] + deg_ref[1] + 1.0)


def _tc1_body(deg_ref, x_ref, w_ref, glo_ref, ghi_ref):
    h = jnp.dot(x_ref[...], w_ref[...], preferred_element_type=jnp.float32,
                precision=lax.Precision.HIGHEST)
    g = h * _dinv(deg_ref)
    glo_ref[...] = g[:, :DH]
    ghi_ref[...] = g[:, DH:]


def _comb1_body(deg_ref, s_ref, glo_ref, ghi_ref, b_ref, w_ref,
                gnlo_ref, gnhi_ref):
    dinv = _dinv(deg_ref)
    tot = jnp.concatenate(
        [s_ref[0, 0] + s_ref[0, 1] + glo_ref[...],
         s_ref[1, 0] + s_ref[1, 1] + ghi_ref[...]],
        axis=1,
    )
    t = tot * dinv + b_ref[...]
    u = jnp.maximum(t, 0.0)
    h = jnp.dot(u, w_ref[...], preferred_element_type=jnp.float32,
                precision=lax.Precision.HIGHEST)
    gn = h * dinv
    gnlo_ref[...] = gn[:, :DH]
    gnhi_ref[...] = gn[:, DH:]


def _comb2_body(deg_ref, s_ref, glo_ref, ghi_ref, b_ref, out_ref):
    tot = jnp.concatenate(
        [s_ref[0, 0] + s_ref[0, 1] + glo_ref[...],
         s_ref[1, 0] + s_ref[1, 1] + ghi_ref[...]],
        axis=1,
    )
    out_ref[...] = tot * _dinv(deg_ref) + b_ref[...]


_tc1 = pl.pallas_call(
    _tc1_body,
    grid=(N // B,),
    in_specs=[_DEG_SPEC, _ROW_SPEC, _MAT_SPEC],
    out_specs=[_HALF_SPEC, _HALF_SPEC],
    out_shape=[
        jax.ShapeDtypeStruct((N, DH), jnp.float32),
        jax.ShapeDtypeStruct((N, DH), jnp.float32),
    ],
)

_comb1 = pl.pallas_call(
    _comb1_body,
    grid=(N // B,),
    in_specs=[_DEG_SPEC, _SH_SPEC, _HALF_SPEC, _HALF_SPEC, _BIAS_SPEC,
              _MAT_SPEC],
    out_specs=[_HALF_SPEC, _HALF_SPEC],
    out_shape=[
        jax.ShapeDtypeStruct((N, DH), jnp.float32),
        jax.ShapeDtypeStruct((N, DH), jnp.float32),
    ],
)

_comb2 = pl.pallas_call(
    _comb2_body,
    grid=(N // B,),
    in_specs=[_DEG_SPEC, _SH_SPEC, _HALF_SPEC, _HALF_SPEC, _BIAS_SPEC],
    out_specs=_ROW_SPEC,
    out_shape=jax.ShapeDtypeStruct((N, D), jnp.float32),
)


def kernel(x, edge_index, W1, b1, W2, b2):
    src = edge_index[0].astype(jnp.int32)
    dst = edge_index[1].astype(jnp.int32)
    epw = E // NW
    padw = E_PAD // NW - epw
    fillw = jnp.arange(padw, dtype=jnp.int32)
    src_pad = jnp.broadcast_to(fillw % 16, (NW, padw))
    dst_pad = jnp.broadcast_to(N + (fillw % TRASH), (NW, padw))
    src2d = jnp.concatenate(
        [src.reshape(NW, epw), src_pad], axis=1).reshape(NCHUNKS, CHUNK)
    dst2d = jnp.concatenate(
        [dst.reshape(NW, epw), dst_pad], axis=1).reshape(NCHUNKS, CHUNK)

    zeros8 = jnp.zeros((NP, DEGW), jnp.float32)
    zerosH = jnp.zeros((NP, DH), jnp.float32)

    deg = _deg_kernel(dst2d, zeros8)
    degc = deg[:, :, :1]

    g1lo, g1hi = _tc1(degc, x, W1)
    s1 = _scatter_half(g1lo, g1hi, src2d, dst2d, zerosH)
    g2lo, g2hi = _comb1(degc, s1, g1lo, g1hi, b1.reshape(1, D), W2)
    s2 = _scatter_half(g2lo, g2hi, src2d, dst2d, zerosH)
    return _comb2(degc, s2, g2lo, g2hi, b2.reshape(1, D))


# speedup vs baseline: 1.0223x; 1.0004x over previous
import functools

import jax
import jax.numpy as jnp
from jax import lax
from jax.experimental import pallas as pl
from jax.experimental.pallas import tpu as pltpu
from jax.experimental.pallas import tpu_sc as plsc

N = 10000
E = 320000
D = 128
DH = D // 2
NC = 2
NS = 16
NW = NC * NS
CHUNK = 128
CPW = 80
NCHUNKS = NW * CPW
E_PAD = NCHUNKS * CHUNK
TRASH = 112
NP = N + TRASH
STRIPE = NP // NS
DEGW = 8

_MESH = plsc.VectorSubcoreMesh(
    core_axis_name="c", subcore_axis_name="s", num_cores=NC, num_subcores=NS
)
_SC_PARAMS = pltpu.CompilerParams(use_tc_tiling_on_sc=False)


def _deg_body(dst2d_hbm, zeros8_hbm, out_hbm, dst_st, ones_st, deg_sh, dsem):
    c = lax.axis_index("c")
    s = lax.axis_index("s")
    wid = s * NC + c
    r0 = s * STRIPE

    pltpu.sync_copy(zeros8_hbm.at[pl.ds(r0, STRIPE)], deg_sh.at[pl.ds(r0, STRIPE)])
    pltpu.sync_copy(dst2d_hbm.at[pl.ds(wid * CPW, CPW)], dst_st)
    for r in range(CHUNK // 2):
        ones_st[pl.ds(2 * r, 2), :] = jnp.ones((2, DEGW), jnp.float32)
    plsc.subcore_barrier()

    k = 8

    def grp(g, carry):
        for b in range(k):
            pltpu.async_copy(
                ones_st, deg_sh.at[dst_st.at[g * k + b]], dsem, add=True
            )
        for b in range(k):
            pltpu.make_async_copy(
                zeros8_hbm.at[pl.ds(0, CHUNK)], ones_st, dsem
            ).wait()
        return carry

    lax.fori_loop(0, CPW // k, grp, 0)
    plsc.subcore_barrier()
    pltpu.sync_copy(
        deg_sh.at[pl.ds(r0, STRIPE)], out_hbm.at[c, pl.ds(r0, STRIPE)]
    )


_deg_kernel = functools.partial(
    pl.kernel,
    out_type=jax.ShapeDtypeStruct((NC, NP, DEGW), jnp.float32),
    mesh=_MESH,
    compiler_params=_SC_PARAMS,
    scratch_types=[
        pltpu.VMEM((CPW, CHUNK), jnp.int32),
        pltpu.VMEM((CHUNK, DEGW), jnp.float32),
        pltpu.VMEM_SHARED((NP, DEGW), jnp.float32),
        pltpu.SemaphoreType.DMA,
    ],
)(_deg_body)


def _make_scatter_body(width, nbuf, npass):
    ngrp = CPW // nbuf

    def body(*refs):
        g_hbms = refs[:npass]
        src2d_hbm, dst2d_hbm, zeros_hbm, out_hbm = refs[npass:npass + 4]
        src_st, dst_st, rowbuf, s_sh, gsem, ssem = refs[npass + 4:]

        c = lax.axis_index("c")
        s = lax.axis_index("s")
        wid = s * NC + c
        r0 = s * STRIPE

        pltpu.sync_copy(src2d_hbm.at[pl.ds(wid * CPW, CPW)], src_st)
        pltpu.sync_copy(dst2d_hbm.at[pl.ds(wid * CPW, CPW)], dst_st)

        for half in range(npass):
            g_hbm = g_hbms[half]
            pltpu.sync_copy(
                zeros_hbm.at[pl.ds(r0, STRIPE)], s_sh.at[pl.ds(r0, STRIPE)]
            )
            plsc.subcore_barrier()

            for b in range(nbuf):
                pltpu.async_copy(
                    g_hbm.at[src_st.at[b]], rowbuf.at[b], gsem.at[b]
                )

            def grp(g, carry):
                for b in range(nbuf):
                    j = g * nbuf + b
                    pltpu.make_async_copy(
                        g_hbm.at[src_st.at[b]], rowbuf.at[b], gsem.at[b]
                    ).wait()
                    pltpu.async_copy(
                        rowbuf.at[b], s_sh.at[dst_st.at[j]], ssem.at[b],
                        add=True,
                    )
                for b in range(nbuf):
                    j2 = (g + 1) * nbuf + b
                    pltpu.make_async_copy(
                        g_hbm.at[src_st.at[b]], rowbuf.at[b], ssem.at[b]
                    ).wait()
                    pltpu.async_copy(
                        g_hbm.at[src_st.at[j2]], rowbuf.at[b], gsem.at[b]
                    )
                return carry

            lax.fori_loop(0, ngrp - 1, grp, 0)

            for b in range(nbuf):
                j = (ngrp - 1) * nbuf + b
                pltpu.make_async_copy(
                    g_hbm.at[src_st.at[b]], rowbuf.at[b], gsem.at[b]
                ).wait()
                pltpu.async_copy(
                    rowbuf.at[b], s_sh.at[dst_st.at[j]], ssem.at[b], add=True
                )
            for b in range(nbuf):
                pltpu.make_async_copy(
                    g_hbm.at[src_st.at[b]], rowbuf.at[b], ssem.at[b]
                ).wait()

            plsc.subcore_barrier()
            pltpu.sync_copy(
                s_sh.at[pl.ds(r0, STRIPE)],
                out_hbm.at[half, c, pl.ds(r0, STRIPE)],
            )

    return body


def _make_scatter_kernel(width, nbuf, npass):
    return functools.partial(
        pl.kernel,
        out_type=jax.ShapeDtypeStruct((npass, NC, NP, width), jnp.float32),
        mesh=_MESH,
        compiler_params=_SC_PARAMS,
        scratch_types=[
            pltpu.VMEM((CPW, CHUNK), jnp.int32),
            pltpu.VMEM((CPW, CHUNK), jnp.int32),
            pltpu.VMEM((nbuf, CHUNK, width), jnp.float32),
            pltpu.VMEM_SHARED((NP, width), jnp.float32),
            pltpu.SemaphoreType.DMA((nbuf,)),
            pltpu.SemaphoreType.DMA((nbuf,)),
        ],
    )(_make_scatter_body(width, nbuf, npass))


_scatter_half = _make_scatter_kernel(DH, 8, 2)


B = 1000

_DEG_SPEC = pl.BlockSpec((NC, B, DEGW), lambda i: (0, i, 0))
_ROW_SPEC = pl.BlockSpec((B, D), lambda i: (i, 0))
_HALF_SPEC = pl.BlockSpec((B, DH), lambda i: (i, 0))
_MAT_SPEC = pl.BlockSpec((D, D), lambda i: (0, 0))
_BIAS_SPEC = pl.BlockSpec((1, D), lambda i: (0, 0))
_SF_SPEC = pl.BlockSpec((1, NC, B, D), lambda i: (0, 0, i, 0))
_SH_SPEC = pl.BlockSpec((2, NC, B, DH), lambda i: (0, 0, i, 0))


def _dinv(deg_ref):
    return lax.rsqrt(deg_ref[0, :, :1] + deg_ref[1, :, :1] + 1.0)


def _tc1_body(deg_ref, x_ref, w_ref, glo_ref, ghi_ref):
    h = jnp.dot(x_ref[...], w_ref[...], preferred_element_type=jnp.float32,
                precision=lax.Precision.HIGHEST)
    g = h * _dinv(deg_ref)
    glo_ref[...] = g[:, :DH]
    ghi_ref[...] = g[:, DH:]


def _comb1_body(deg_ref, s_ref, glo_ref, ghi_ref, b_ref, w_ref,
                gnlo_ref, gnhi_ref):
    dinv = _dinv(deg_ref)
    tot = jnp.concatenate(
        [s_ref[0, 0] + s_ref[0, 1] + glo_ref[...],
         s_ref[1, 0] + s_ref[1, 1] + ghi_ref[...]],
        axis=1,
    )
    t = tot * dinv + b_ref[...]
    u = jnp.maximum(t, 0.0)
    h = jnp.dot(u, w_ref[...], preferred_element_type=jnp.float32,
                precision=lax.Precision.HIGHEST)
    gn = h * dinv
    gnlo_ref[...] = gn[:, :DH]
    gnhi_ref[...] = gn[:, DH:]


def _comb2_body(deg_ref, s_ref, glo_ref, ghi_ref, b_ref, out_ref):
    tot = jnp.concatenate(
        [s_ref[0, 0] + s_ref[0, 1] + glo_ref[...],
         s_ref[1, 0] + s_ref[1, 1] + ghi_ref[...]],
        axis=1,
    )
    out_ref[...] = tot * _dinv(deg_ref) + b_ref[...]


_tc1 = pl.pallas_call(
    _tc1_body,
    grid=(N // B,),
    in_specs=[_DEG_SPEC, _ROW_SPEC, _MAT_SPEC],
    out_specs=[_HALF_SPEC, _HALF_SPEC],
    out_shape=[
        jax.ShapeDtypeStruct((N, DH), jnp.float32),
        jax.ShapeDtypeStruct((N, DH), jnp.float32),
    ],
)

_comb1 = pl.pallas_call(
    _comb1_body,
    grid=(N // B,),
    in_specs=[_DEG_SPEC, _SH_SPEC, _HALF_SPEC, _HALF_SPEC, _BIAS_SPEC,
              _MAT_SPEC],
    out_specs=[_HALF_SPEC, _HALF_SPEC],
    out_shape=[
        jax.ShapeDtypeStruct((N, DH), jnp.float32),
        jax.ShapeDtypeStruct((N, DH), jnp.float32),
    ],
)

_comb2 = pl.pallas_call(
    _comb2_body,
    grid=(N // B,),
    in_specs=[_DEG_SPEC, _SH_SPEC, _HALF_SPEC, _HALF_SPEC, _BIAS_SPEC],
    out_specs=_ROW_SPEC,
    out_shape=jax.ShapeDtypeStruct((N, D), jnp.float32),
)


def kernel(x, edge_index, W1, b1, W2, b2):
    src = edge_index[0].astype(jnp.int32)
    dst = edge_index[1].astype(jnp.int32)
    epw = E // NW
    padw = E_PAD // NW - epw
    fillw = jnp.arange(padw, dtype=jnp.int32)
    src_pad = jnp.broadcast_to(fillw % 16, (NW, padw))
    dst_pad = jnp.broadcast_to(N + (fillw % TRASH), (NW, padw))
    src2d = jnp.concatenate(
        [src.reshape(NW, epw), src_pad], axis=1).reshape(NCHUNKS, CHUNK)
    dst2d = jnp.concatenate(
        [dst.reshape(NW, epw), dst_pad], axis=1).reshape(NCHUNKS, CHUNK)

    zeros8 = jnp.zeros((NP, DEGW), jnp.float32)
    zerosH = jnp.zeros((NP, DH), jnp.float32)

    degc = _deg_kernel(dst2d, zeros8)

    g1lo, g1hi = _tc1(degc, x, W1)
    s1 = _scatter_half(g1lo, g1hi, src2d, dst2d, zerosH)
    g2lo, g2hi = _comb1(degc, s1, g1lo, g1hi, b1.reshape(1, D), W2)
    s2 = _scatter_half(g2lo, g2hi, src2d, dst2d, zerosH)
    return _comb2(degc, s2, g2lo, g2hi, b2.reshape(1, D))
